# baseline (speedup 1.0000x reference)
import functools

import jax
import jax.numpy as jnp
from jax import lax
from jax.experimental import pallas as pl
from jax.experimental.pallas import tpu as pltpu
from jax.experimental.pallas import tpu_sc as plsc

DIM = 64
CB = 256
NCB = 16
NLOG = CB * NCB
TOK = 16384
BT = 2048

_NC, _NS, _L = 2, 16, 16
NW = _NC * _NS
TPW = TOK // NW
TCH = 32
NCHUNK = TPW // TCH
ROWS = TCH * NCB
IDXW = 128
NGATH = ROWS // IDXW


def _tc_body(x_ref, w_ref, b_ref, idx_ref, off_ref):
    xt = x_ref[...]
    w = w_ref[...]
    logits = lax.dot_general(
        w, xt, (((1,), (1,)), ((), ())), preferred_element_type=jnp.float32)
    logits = logits + b_ref[...]
    iota0 = lax.broadcasted_iota(jnp.int32, (CB, BT), 0)
    cols = []
    for j in range(NCB):
        blk = logits[j * CB:(j + 1) * CB, :]
        m = jnp.max(blk, axis=0)
        cand = jnp.where(blk == m[None, :], iota0, CB)
        idxj = jnp.min(cand, axis=0).astype(jnp.int32)
        cols.append(idxj[None, :])
    idx_t = jnp.concatenate(cols, axis=0)
    idx = idx_t.T
    joff = lax.broadcasted_iota(jnp.int32, (1, NCB), 1) * CB
    idx_ref[...] = idx
    off_ref[...] = idx + joff


def _tc_encode(x, W, b, base_step, nsteps):
    b2 = b.reshape(NLOG, 1)
    ntok = nsteps * BT
    return pl.pallas_call(
        _tc_body,
        grid=(nsteps,),
        in_specs=[
            pl.BlockSpec((BT, DIM), lambda i: (i + base_step, 0)),
            pl.BlockSpec((NLOG, DIM), lambda i: (0, 0)),
            pl.BlockSpec((NLOG, 1), lambda i: (0, 0)),
        ],
        out_specs=[
            pl.BlockSpec((BT, NCB), lambda i: (i, 0)),
            pl.BlockSpec((BT, NCB), lambda i: (i, 0)),
        ],
        out_shape=[
            jax.ShapeDtypeStruct((ntok, NCB), jnp.int32),
            jax.ShapeDtypeStruct((ntok, NCB), jnp.int32),
        ],
    )(x, W, b2)


@functools.lru_cache(maxsize=4)
def _sc_gather_fn(ntok):
    tpw = ntok // NW
    return functools.partial(
        pl.kernel,
        mesh=plsc.VectorSubcoreMesh(
            core_axis_name="c", subcore_axis_name="s",
            num_cores=_NC, num_subcores=_NS),
        out_type=jax.ShapeDtypeStruct((ntok, DIM), jnp.float32),
        compiler_params=pltpu.CompilerParams(use_tc_tiling_on_sc=False),
        scratch_types=[
            pltpu.VMEM((tpw, NCB), jnp.int32),
            pltpu.VMEM((tpw * NCB,), jnp.int32),
            pltpu.VMEM((ROWS, DIM), jnp.float32),
            pltpu.VMEM((ROWS, DIM), jnp.float32),
            pltpu.VMEM((TCH, DIM), jnp.float32),
            pltpu.VMEM((TCH, DIM), jnp.float32),
            pltpu.SemaphoreType.DMA,
            pltpu.SemaphoreType.DMA,
            pltpu.SemaphoreType.DMA,
        ],
    )(functools.partial(_sc_gather_body, tpw))


def _sc_gather_body(TPW, off_hbm, cent_hbm, out_hbm, idx2_v, idxf_v, rows0_v,
                    rows1_v, acc0_v, acc1_v, gsem0, gsem1, osem):
    NCHUNK = TPW // TCH
    wid = lax.axis_index("s") * _NC + lax.axis_index("c")
    base = pl.multiple_of(wid * TPW, TPW)
    gsems = (gsem0, gsem1)
    rows_bufs = (rows0_v, rows1_v)
    acc_bufs = (acc0_v, acc1_v)

    pltpu.sync_copy(off_hbm.at[pl.ds(base, TPW)], idx2_v)

    def flat_body(t, carry):
        for tt in range(8):
            idxf_v[pl.ds((t * 8 + tt) * NCB, NCB)] = idx2_v[t * 8 + tt, :]
        return carry

    lax.fori_loop(0, TPW // 8, flat_body, 0)

    def fire(c, buf):
        c0 = pl.multiple_of(c * ROWS, ROWS)
        for g in range(NGATH):
            pltpu.async_copy(
                cent_hbm.at[idxf_v.at[pl.ds(c0 + g * IDXW, IDXW)]],
                rows_bufs[buf].at[pl.ds(g * IDXW, IDXW)],
                gsems[buf],
            )

    def drain(buf):
        pltpu.make_async_copy(
            cent_hbm.at[pl.ds(0, ROWS)], rows_bufs[buf], gsems[buf]).wait()

    def reduce_and_write(c, buf):
        rows_v = rows_bufs[buf]
        acc_v = acc_bufs[buf]

        def tok_body(t, carry):
            for tt in range(4):
                r0 = (t * 4 + tt) * NCB
                for k in range(DIM // _L):
                    s = rows_v[r0, pl.ds(k * _L, _L)]
                    for j in range(1, NCB):
                        s = s + rows_v[r0 + j, pl.ds(k * _L, _L)]
                    acc_v[t * 4 + tt, pl.ds(k * _L, _L)] = s
            return carry

        lax.fori_loop(0, TCH // 4, tok_body, 0)
        out0 = pl.multiple_of(base + c * TCH, TCH)
        pltpu.sync_copy(acc_v, out_hbm.at[pl.ds(out0, TCH)])

    fire(0, 0)

    def pair_body(i, carry):
        c0 = i * 2
        fire(c0 + 1, 1)
        drain(0)
        reduce_and_write(c0, 0)
        fire(c0 + 2, 0)
        drain(1)
        reduce_and_write(c0 + 1, 1)
        return carry

    lax.fori_loop(0, NCHUNK // 2 - 1, pair_body, 0)
    fire(NCHUNK - 1, 1)
    drain(0)
    reduce_and_write(NCHUNK - 2, 0)
    drain(1)
    reduce_and_write(NCHUNK - 1, 1)


def kernel(x, W, b, centers):
    split = (8, 8)
    idxs, recs = [], []
    base = 0
    for steps in split:
        idx_s, off_s = _tc_encode(x, W, b, base, steps)
        rec_s = _sc_gather_fn(steps * BT)(off_s, centers)
        idxs.append(idx_s)
        recs.append(rec_s)
        base += steps
    indexes = jnp.concatenate(idxs, axis=0)
    recon = jnp.concatenate(recs, axis=0)
    return indexes, recon


# speedup vs baseline: 1.8423x; 1.8423x over previous
import functools

import jax
import jax.numpy as jnp
from jax import lax
from jax.experimental import pallas as pl
from jax.experimental.pallas import tpu as pltpu
from jax.experimental.pallas import tpu_sc as plsc

DIM = 64
CB = 256
NCB = 16
NLOG = CB * NCB
TOK = 16384
BT = 1024

_NC, _NS, _L = 2, 16, 16
NW = _NC * _NS
TPW = TOK // NW
TCH = 32
NCHUNK = TPW // TCH
ROWS = TCH * NCB
IDXW = 128
NGATH = ROWS // IDXW


def _tc_body(x_ref, w_ref, b_ref, idx_ref, off_ref):
    xt = x_ref[...]
    w = w_ref[...]
    logits = lax.dot_general(
        w, xt, (((1,), (1,)), ((), ())), preferred_element_type=jnp.float32)
    logits = logits + b_ref[...]
    iota0 = lax.broadcasted_iota(jnp.int32, (CB, BT), 0)
    cols = []
    for j in range(NCB):
        blk = logits[j * CB:(j + 1) * CB, :]
        m = jnp.max(blk, axis=0)
        cand = jnp.where(blk == m[None, :], iota0, CB)
        idxj = jnp.min(cand, axis=0).astype(jnp.int32)
        cols.append(idxj[None, :])
    idx_t = jnp.concatenate(cols, axis=0)
    idx = idx_t.T
    joff = lax.broadcasted_iota(jnp.int32, (1, NCB), 1) * CB
    idx_ref[...] = idx
    off_ref[...] = idx + joff


def _tc_encode(x, W, b, base_step, nsteps):
    b2 = b.reshape(NLOG, 1)
    ntok = nsteps * BT
    return pl.pallas_call(
        _tc_body,
        grid=(nsteps,),
        in_specs=[
            pl.BlockSpec((BT, DIM), lambda i: (i + base_step, 0)),
            pl.BlockSpec((NLOG, DIM), lambda i: (0, 0)),
            pl.BlockSpec((NLOG, 1), lambda i: (0, 0)),
        ],
        out_specs=[
            pl.BlockSpec((BT, NCB), lambda i: (i, 0)),
            pl.BlockSpec((BT, NCB), lambda i: (i, 0)),
        ],
        out_shape=[
            jax.ShapeDtypeStruct((ntok, NCB), jnp.int32),
            jax.ShapeDtypeStruct((ntok, NCB), jnp.int32),
        ],
    )(x, W, b2)


@functools.lru_cache(maxsize=4)
def _sc_gather_fn(ntok):
    tpw = ntok // NW
    return functools.partial(
        pl.kernel,
        mesh=plsc.VectorSubcoreMesh(
            core_axis_name="c", subcore_axis_name="s",
            num_cores=_NC, num_subcores=_NS),
        out_type=jax.ShapeDtypeStruct((ntok, DIM), jnp.float32),
        compiler_params=pltpu.CompilerParams(use_tc_tiling_on_sc=False),
        scratch_types=[
            pltpu.VMEM((tpw, NCB), jnp.int32),
            pltpu.VMEM((tpw * NCB,), jnp.int32),
            pltpu.VMEM((ROWS, DIM), jnp.float32),
            pltpu.VMEM((ROWS, DIM), jnp.float32),
            pltpu.VMEM((TCH, DIM), jnp.float32),
            pltpu.VMEM((TCH, DIM), jnp.float32),
            pltpu.SemaphoreType.DMA,
            pltpu.SemaphoreType.DMA,
            pltpu.SemaphoreType.DMA,
        ],
    )(functools.partial(_sc_gather_body, tpw))


def _sc_gather_body(TPW, off_hbm, cent_hbm, out_hbm, idx2_v, idxf_v, rows0_v,
                    rows1_v, acc0_v, acc1_v, gsem0, gsem1, osem):
    NCHUNK = TPW // TCH
    wid = lax.axis_index("s") * _NC + lax.axis_index("c")
    base = pl.multiple_of(wid * TPW, TPW)
    gsems = (gsem0, gsem1)
    rows_bufs = (rows0_v, rows1_v)
    acc_bufs = (acc0_v, acc1_v)

    pltpu.sync_copy(off_hbm.at[pl.ds(base, TPW)], idx2_v)

    def flat_body(t, carry):
        for tt in range(8):
            idxf_v[pl.ds((t * 8 + tt) * NCB, NCB)] = idx2_v[t * 8 + tt, :]
        return carry

    lax.fori_loop(0, TPW // 8, flat_body, 0)

    def fire(c, buf):
        c0 = pl.multiple_of(c * ROWS, ROWS)
        for g in range(NGATH):
            pltpu.async_copy(
                cent_hbm.at[idxf_v.at[pl.ds(c0 + g * IDXW, IDXW)]],
                rows_bufs[buf].at[pl.ds(g * IDXW, IDXW)],
                gsems[buf],
            )

    def drain(buf):
        pltpu.make_async_copy(
            cent_hbm.at[pl.ds(0, ROWS)], rows_bufs[buf], gsems[buf]).wait()

    def reduce_and_write(c, buf):
        rows_v = rows_bufs[buf]
        acc_v = acc_bufs[buf]

        def tok_body(t, carry):
            for tt in range(4):
                r0 = (t * 4 + tt) * NCB
                for k in range(DIM // _L):
                    s = rows_v[r0, pl.ds(k * _L, _L)]
                    for j in range(1, NCB):
                        s = s + rows_v[r0 + j, pl.ds(k * _L, _L)]
                    acc_v[t * 4 + tt, pl.ds(k * _L, _L)] = s
            return carry

        lax.fori_loop(0, TCH // 4, tok_body, 0)
        out0 = pl.multiple_of(base + c * TCH, TCH)
        pltpu.sync_copy(acc_v, out_hbm.at[pl.ds(out0, TCH)])

    fire(0, 0)

    def pair_body(i, carry):
        c0 = i * 2
        fire(c0 + 1, 1)
        drain(0)
        reduce_and_write(c0, 0)
        fire(c0 + 2, 0)
        drain(1)
        reduce_and_write(c0 + 1, 1)
        return carry

    lax.fori_loop(0, NCHUNK // 2 - 1, pair_body, 0)
    fire(NCHUNK - 1, 1)
    drain(0)
    reduce_and_write(NCHUNK - 2, 0)
    drain(1)
    reduce_and_write(NCHUNK - 1, 1)


def kernel(x, W, b, centers):
    half = TOK // BT // 2
    split = (half, half)
    idxs, recs = [], []
    base = 0
    for steps in split:
        idx_s, off_s = _tc_encode(x, W, b, base, steps)
        rec_s = _sc_gather_fn(steps * BT)(off_s, centers)
        idxs.append(idx_s)
        recs.append(rec_s)
        base += steps
    indexes = jnp.concatenate(idxs, axis=0)
    recon = jnp.concatenate(recs, axis=0)
    return indexes, recon
